# Initial kernel scaffold; baseline (speedup 1.0000x reference)
import jax
import jax.numpy as jnp
from jax.experimental import pallas as pl


def kernel(input_ids, emb, Ws, bs, We, be):
    raise NotImplementedError("write your pallas kernel here")


# speedup vs baseline: 18.6973x; 18.6973x over previous
import functools

import jax
import jax.numpy as jnp
from jax import lax
from jax.experimental import pallas as pl
from jax.experimental.pallas import tpu as pltpu
from jax.experimental.pallas import tpu_sc as plsc

_NUM_SC = 2
_NUM_SUBCORES = 16

_TABLE_BLOCK = 4096
_CHUNK = 12800


def _table_body(emb_ref, w_ref, b_ref, out_ref):
    out_ref[...] = (
        jnp.dot(emb_ref[...], w_ref[...], preferred_element_type=jnp.float32)
        + b_ref[...]
    )


def _build_table(emb, w2, b2):
    v, h = emb.shape
    grid = pl.cdiv(v, _TABLE_BLOCK)
    return pl.pallas_call(
        _table_body,
        grid=(grid,),
        in_specs=[
            pl.BlockSpec((_TABLE_BLOCK, h), lambda i: (i, 0)),
            pl.BlockSpec((h, 2), lambda i: (0, 0)),
            pl.BlockSpec((1, 2), lambda i: (0, 0)),
        ],
        out_specs=pl.BlockSpec((_TABLE_BLOCK, 2), lambda i: (i, 0)),
        out_shape=jax.ShapeDtypeStruct((v, 2), jnp.float32),
    )(emb, w2, b2)


def _gather_logits(tab_s, tab_e, idx):
    n = idx.shape[0]
    n_per_tile = n // _NUM_SUBCORES
    assert n_per_tile % _CHUNK == 0 and _CHUNK % 16 == 0
    n_chunks = n_per_tile // _CHUNK
    mesh = plsc.VectorSubcoreMesh(
        core_axis_name="c",
        subcore_axis_name="s",
        num_cores=_NUM_SC,
        num_subcores=_NUM_SUBCORES,
    )

    @functools.partial(
        pl.kernel,
        out_type=(
            jax.ShapeDtypeStruct((n,), jnp.float32),
            jax.ShapeDtypeStruct((n,), jnp.float32),
        ),
        mesh=mesh,
        scratch_types=[
            pltpu.VMEM((tab_s.shape[0],), jnp.float32),
            pltpu.VMEM((_CHUNK,), jnp.int32),
            pltpu.VMEM((_CHUNK,), jnp.float32),
        ],
        compiler_params=pltpu.CompilerParams(needs_layout_passes=False),
    )
    def k(tabs_hbm, tabe_hbm, idx_hbm, outs_hbm, oute_hbm, tab_v, idx_v, out_v):
        c = lax.axis_index("c")
        s = lax.axis_index("s")
        base = s * n_per_tile

        @pl.when(c == 0)
        def _():
            pltpu.sync_copy(tabs_hbm, tab_v)

        @pl.when(c == 1)
        def _():
            pltpu.sync_copy(tabe_hbm, tab_v)

        for chunk in range(n_chunks):
            off = base + chunk * _CHUNK
            pltpu.sync_copy(idx_hbm.at[pl.ds(off, _CHUNK)], idx_v)

            def body(j, _):
                vals = plsc.load_gather(tab_v, [idx_v[pl.ds(j * 16, 16)]])
                out_v[pl.ds(j * 16, 16)] = vals
                return 0

            lax.fori_loop(0, _CHUNK // 16, body, 0)

            @pl.when(c == 0)
            def _():
                pltpu.sync_copy(out_v, outs_hbm.at[pl.ds(off, _CHUNK)])

            @pl.when(c == 1)
            def _():
                pltpu.sync_copy(out_v, oute_hbm.at[pl.ds(off, _CHUNK)])

    return k(tab_s, tab_e, idx)


def kernel(input_ids, emb, Ws, bs, We, be):
    b, l = input_ids.shape
    w2 = jnp.concatenate([Ws, We], axis=1)
    b2 = jnp.stack([bs[0], be[0]])[None, :]
    table = _build_table(emb, w2, b2)
    idx = input_ids.reshape(-1).astype(jnp.int32)
    start_flat, end_flat = _gather_logits(table[:, 0], table[:, 1], idx)
    return (start_flat.reshape(b, l), end_flat.reshape(b, l))


# speedup vs baseline: 26.1271x; 1.3974x over previous
import functools

import jax
import jax.numpy as jnp
from jax import lax
from jax.experimental import pallas as pl
from jax.experimental.pallas import tpu as pltpu
from jax.experimental.pallas import tpu_sc as plsc

_NUM_SC = 2
_NUM_SUBCORES = 16

_TABLE_BLOCK = 4096
_CHUNK = 12800


def _table_body(emb_ref, ws_ref, we_ref, bs_ref, be_ref, outs_ref, oute_ref):
    emb_blk = emb_ref[...]
    dn = (((1,), (1,)), ((), ()))
    outs_ref[...] = (
        lax.dot_general(ws_ref[...], emb_blk, dn, preferred_element_type=jnp.float32)
        + bs_ref[0, 0]
    )
    oute_ref[...] = (
        lax.dot_general(we_ref[...], emb_blk, dn, preferred_element_type=jnp.float32)
        + be_ref[0, 0]
    )


def _build_tables(emb, ws_row, we_row, bs, be):
    v, h = emb.shape
    grid = pl.cdiv(v, _TABLE_BLOCK)
    return pl.pallas_call(
        _table_body,
        grid=(grid,),
        in_specs=[
            pl.BlockSpec((_TABLE_BLOCK, h), lambda i: (i, 0)),
            pl.BlockSpec((1, h), lambda i: (0, 0)),
            pl.BlockSpec((1, h), lambda i: (0, 0)),
            pl.BlockSpec((1, 1), lambda i: (0, 0)),
            pl.BlockSpec((1, 1), lambda i: (0, 0)),
        ],
        out_specs=[
            pl.BlockSpec((1, _TABLE_BLOCK), lambda i: (0, i)),
            pl.BlockSpec((1, _TABLE_BLOCK), lambda i: (0, i)),
        ],
        out_shape=[
            jax.ShapeDtypeStruct((1, v), jnp.float32),
            jax.ShapeDtypeStruct((1, v), jnp.float32),
        ],
    )(emb, ws_row, we_row, bs, be)


def _gather_logits(tab_s, tab_e, idx):
    n = idx.shape[0]
    n_per_tile = n // _NUM_SUBCORES
    assert n_per_tile % _CHUNK == 0 and _CHUNK % 16 == 0
    n_chunks = n_per_tile // _CHUNK
    mesh = plsc.VectorSubcoreMesh(
        core_axis_name="c",
        subcore_axis_name="s",
        num_cores=_NUM_SC,
        num_subcores=_NUM_SUBCORES,
    )

    @functools.partial(
        pl.kernel,
        out_type=(
            jax.ShapeDtypeStruct((n,), jnp.float32),
            jax.ShapeDtypeStruct((n,), jnp.float32),
        ),
        mesh=mesh,
        scratch_types=[
            pltpu.VMEM((tab_s.shape[0],), jnp.float32),
            pltpu.VMEM((_CHUNK,), jnp.int32),
            pltpu.VMEM((_CHUNK,), jnp.float32),
        ],
        compiler_params=pltpu.CompilerParams(needs_layout_passes=False),
    )
    def k(tabs_hbm, tabe_hbm, idx_hbm, outs_hbm, oute_hbm, tab_v, idx_v, out_v):
        c = lax.axis_index("c")
        s = lax.axis_index("s")
        base = s * n_per_tile

        @pl.when(c == 0)
        def _():
            pltpu.sync_copy(tabs_hbm, tab_v)

        @pl.when(c == 1)
        def _():
            pltpu.sync_copy(tabe_hbm, tab_v)

        for chunk in range(n_chunks):
            off = base + chunk * _CHUNK
            pltpu.sync_copy(idx_hbm.at[pl.ds(off, _CHUNK)], idx_v)

            def body(j, _):
                vals = plsc.load_gather(tab_v, [idx_v[pl.ds(j * 16, 16)]])
                out_v[pl.ds(j * 16, 16)] = vals
                return 0

            lax.fori_loop(0, _CHUNK // 16, body, 0)

            @pl.when(c == 0)
            def _():
                pltpu.sync_copy(out_v, outs_hbm.at[pl.ds(off, _CHUNK)])

            @pl.when(c == 1)
            def _():
                pltpu.sync_copy(out_v, oute_hbm.at[pl.ds(off, _CHUNK)])

    return k(tab_s, tab_e, idx)


def kernel(input_ids, emb, Ws, bs, We, be):
    b, l = input_ids.shape
    v = emb.shape[0]
    tab_s, tab_e = _build_tables(
        emb,
        Ws.reshape(1, -1),
        We.reshape(1, -1),
        bs.reshape(1, 1),
        be.reshape(1, 1),
    )
    idx = input_ids.reshape(-1).astype(jnp.int32)
    start_flat, end_flat = _gather_logits(tab_s.reshape(v), tab_e.reshape(v), idx)
    return (start_flat.reshape(b, l), end_flat.reshape(b, l))


# speedup vs baseline: 27.2327x; 1.0423x over previous
import functools

import jax
import jax.numpy as jnp
from jax import lax
from jax.experimental import pallas as pl
from jax.experimental.pallas import tpu as pltpu
from jax.experimental.pallas import tpu_sc as plsc

_NUM_SC = 2
_NUM_SUBCORES = 16

_TABLE_BLOCK = 4096
_CHUNK = 12800


def _table_body(emb_ref, ws_ref, we_ref, bs_ref, be_ref, outs_ref, oute_ref):
    emb_blk = emb_ref[...]
    dn = (((1,), (1,)), ((), ()))
    outs_ref[...] = (
        lax.dot_general(ws_ref[...], emb_blk, dn, preferred_element_type=jnp.float32)
        + bs_ref[0, 0]
    ).reshape(emb_blk.shape[0])
    oute_ref[...] = (
        lax.dot_general(we_ref[...], emb_blk, dn, preferred_element_type=jnp.float32)
        + be_ref[0, 0]
    ).reshape(emb_blk.shape[0])


def _build_tables(emb, ws_row, we_row, bs, be):
    v, h = emb.shape
    grid = pl.cdiv(v, _TABLE_BLOCK)
    return pl.pallas_call(
        _table_body,
        grid=(grid,),
        in_specs=[
            pl.BlockSpec((_TABLE_BLOCK, h), lambda i: (i, 0)),
            pl.BlockSpec((1, h), lambda i: (0, 0)),
            pl.BlockSpec((1, h), lambda i: (0, 0)),
            pl.BlockSpec((1, 1), lambda i: (0, 0)),
            pl.BlockSpec((1, 1), lambda i: (0, 0)),
        ],
        out_specs=[
            pl.BlockSpec((_TABLE_BLOCK,), lambda i: (i,)),
            pl.BlockSpec((_TABLE_BLOCK,), lambda i: (i,)),
        ],
        out_shape=[
            jax.ShapeDtypeStruct((v,), jnp.float32),
            jax.ShapeDtypeStruct((v,), jnp.float32),
        ],
    )(emb, ws_row, we_row, bs, be)


def _gather_logits(tab_s, tab_e, idx):
    n = idx.shape[0]
    n_per_tile = n // _NUM_SUBCORES
    assert n_per_tile % _CHUNK == 0 and _CHUNK % 16 == 0
    n_chunks = n_per_tile // _CHUNK
    mesh = plsc.VectorSubcoreMesh(
        core_axis_name="c",
        subcore_axis_name="s",
        num_cores=_NUM_SC,
        num_subcores=_NUM_SUBCORES,
    )

    @functools.partial(
        pl.kernel,
        out_type=(
            jax.ShapeDtypeStruct((n,), jnp.float32),
            jax.ShapeDtypeStruct((n,), jnp.float32),
        ),
        mesh=mesh,
        scratch_types=[
            pltpu.VMEM((tab_s.shape[0],), jnp.float32),
            pltpu.VMEM((_CHUNK,), jnp.int32),
            pltpu.VMEM((_CHUNK,), jnp.float32),
        ],
        compiler_params=pltpu.CompilerParams(needs_layout_passes=False),
    )
    def k(tabs_hbm, tabe_hbm, idx_hbm, outs_hbm, oute_hbm, tab_v, idx_v, out_v):
        c = lax.axis_index("c")
        s = lax.axis_index("s")
        base = s * n_per_tile

        @pl.when(c == 0)
        def _():
            pltpu.sync_copy(tabs_hbm, tab_v)

        @pl.when(c == 1)
        def _():
            pltpu.sync_copy(tabe_hbm, tab_v)

        for chunk in range(n_chunks):
            off = base + chunk * _CHUNK
            pltpu.sync_copy(idx_hbm.at[pl.ds(off, _CHUNK)], idx_v)

            def body(j, _):
                vals = plsc.load_gather(tab_v, [idx_v[pl.ds(j * 16, 16)]])
                out_v[pl.ds(j * 16, 16)] = vals
                return 0

            lax.fori_loop(0, _CHUNK // 16, body, 0)

            @pl.when(c == 0)
            def _():
                pltpu.sync_copy(out_v, outs_hbm.at[pl.ds(off, _CHUNK)])

            @pl.when(c == 1)
            def _():
                pltpu.sync_copy(out_v, oute_hbm.at[pl.ds(off, _CHUNK)])

    return k(tab_s, tab_e, idx)


def kernel(input_ids, emb, Ws, bs, We, be):
    b, l = input_ids.shape
    v = emb.shape[0]
    tab_s, tab_e = _build_tables(
        emb,
        Ws.reshape(1, -1),
        We.reshape(1, -1),
        bs.reshape(1, 1),
        be.reshape(1, 1),
    )
    idx = input_ids.reshape(-1).astype(jnp.int32)
    start_flat, end_flat = _gather_logits(tab_s, tab_e, idx)
    return (start_flat.reshape(b, l), end_flat.reshape(b, l))


# speedup vs baseline: 28.7983x; 1.0575x over previous
import functools

import jax
import jax.numpy as jnp
from jax import lax
from jax.experimental import pallas as pl
from jax.experimental.pallas import tpu as pltpu
from jax.experimental.pallas import tpu_sc as plsc

_NUM_SC = 2
_NUM_SUBCORES = 16

_TABLE_BLOCK = 8192
_CHUNK = 12800


def _table_body(emb_ref, ws_ref, we_ref, bs_ref, be_ref, outs_ref, oute_ref):
    emb_blk = emb_ref[...]
    dn = (((1,), (1,)), ((), ()))
    outs_ref[...] = (
        lax.dot_general(ws_ref[...], emb_blk, dn, preferred_element_type=jnp.float32)
        + bs_ref[0, 0]
    ).reshape(emb_blk.shape[0])
    oute_ref[...] = (
        lax.dot_general(we_ref[...], emb_blk, dn, preferred_element_type=jnp.float32)
        + be_ref[0, 0]
    ).reshape(emb_blk.shape[0])


def _build_tables(emb, ws_row, we_row, bs, be):
    v, h = emb.shape
    grid = pl.cdiv(v, _TABLE_BLOCK)
    return pl.pallas_call(
        _table_body,
        grid=(grid,),
        in_specs=[
            pl.BlockSpec((_TABLE_BLOCK, h), lambda i: (i, 0)),
            pl.BlockSpec((1, h), lambda i: (0, 0)),
            pl.BlockSpec((1, h), lambda i: (0, 0)),
            pl.BlockSpec((1, 1), lambda i: (0, 0)),
            pl.BlockSpec((1, 1), lambda i: (0, 0)),
        ],
        out_specs=[
            pl.BlockSpec((_TABLE_BLOCK,), lambda i: (i,)),
            pl.BlockSpec((_TABLE_BLOCK,), lambda i: (i,)),
        ],
        out_shape=[
            jax.ShapeDtypeStruct((v,), jnp.float32),
            jax.ShapeDtypeStruct((v,), jnp.float32),
        ],
    )(emb, ws_row, we_row, bs, be)


def _gather_logits(tab_s, tab_e, idx):
    n = idx.shape[0]
    n_per_tile = n // _NUM_SUBCORES
    assert n_per_tile % _CHUNK == 0 and _CHUNK % 16 == 0
    n_chunks = n_per_tile // _CHUNK
    mesh = plsc.VectorSubcoreMesh(
        core_axis_name="c",
        subcore_axis_name="s",
        num_cores=_NUM_SC,
        num_subcores=_NUM_SUBCORES,
    )

    @functools.partial(
        pl.kernel,
        out_type=(
            jax.ShapeDtypeStruct((n,), jnp.float32),
            jax.ShapeDtypeStruct((n,), jnp.float32),
        ),
        mesh=mesh,
        scratch_types=[
            pltpu.VMEM((tab_s.shape[0],), jnp.float32),
            pltpu.VMEM((_CHUNK,), jnp.int32),
            pltpu.VMEM((_CHUNK,), jnp.float32),
        ],
        compiler_params=pltpu.CompilerParams(needs_layout_passes=False),
    )
    def k(tabs_hbm, tabe_hbm, idx_hbm, outs_hbm, oute_hbm, tab_v, idx_v, out_v):
        c = lax.axis_index("c")
        s = lax.axis_index("s")
        base = s * n_per_tile

        @pl.when(c == 0)
        def _():
            pltpu.sync_copy(tabs_hbm, tab_v)

        @pl.when(c == 1)
        def _():
            pltpu.sync_copy(tabe_hbm, tab_v)

        for chunk in range(n_chunks):
            off = base + chunk * _CHUNK
            pltpu.sync_copy(idx_hbm.at[pl.ds(off, _CHUNK)], idx_v)

            def body(j, _):
                vals = plsc.load_gather(tab_v, [idx_v[pl.ds(j * 16, 16)]])
                out_v[pl.ds(j * 16, 16)] = vals
                return 0

            lax.fori_loop(0, _CHUNK // 16, body, 0)

            @pl.when(c == 0)
            def _():
                pltpu.sync_copy(out_v, outs_hbm.at[pl.ds(off, _CHUNK)])

            @pl.when(c == 1)
            def _():
                pltpu.sync_copy(out_v, oute_hbm.at[pl.ds(off, _CHUNK)])

    return k(tab_s, tab_e, idx)


def kernel(input_ids, emb, Ws, bs, We, be):
    b, l = input_ids.shape
    v = emb.shape[0]
    tab_s, tab_e = _build_tables(
        emb,
        Ws.reshape(1, -1),
        We.reshape(1, -1),
        bs.reshape(1, 1),
        be.reshape(1, 1),
    )
    idx = input_ids.reshape(-1).astype(jnp.int32)
    start_flat, end_flat = _gather_logits(tab_s, tab_e, idx)
    return (start_flat.reshape(b, l), end_flat.reshape(b, l))


# speedup vs baseline: 29.2114x; 1.0143x over previous
import functools

import jax
import jax.numpy as jnp
from jax import lax
from jax.experimental import pallas as pl
from jax.experimental.pallas import tpu as pltpu
from jax.experimental.pallas import tpu_sc as plsc

_NUM_SC = 2
_NUM_SUBCORES = 16

_TABLE_BLOCK = 16384
_CHUNK = 12800


def _table_body(emb_ref, ws_ref, we_ref, bs_ref, be_ref, outs_ref, oute_ref):
    emb_blk = emb_ref[...]
    dn = (((1,), (1,)), ((), ()))
    outs_ref[...] = (
        lax.dot_general(ws_ref[...], emb_blk, dn, preferred_element_type=jnp.float32)
        + bs_ref[0, 0]
    ).reshape(emb_blk.shape[0])
    oute_ref[...] = (
        lax.dot_general(we_ref[...], emb_blk, dn, preferred_element_type=jnp.float32)
        + be_ref[0, 0]
    ).reshape(emb_blk.shape[0])


def _build_tables(emb, ws_row, we_row, bs, be):
    v, h = emb.shape
    grid = pl.cdiv(v, _TABLE_BLOCK)
    return pl.pallas_call(
        _table_body,
        grid=(grid,),
        in_specs=[
            pl.BlockSpec((_TABLE_BLOCK, h), lambda i: (i, 0)),
            pl.BlockSpec((1, h), lambda i: (0, 0)),
            pl.BlockSpec((1, h), lambda i: (0, 0)),
            pl.BlockSpec((1, 1), lambda i: (0, 0)),
            pl.BlockSpec((1, 1), lambda i: (0, 0)),
        ],
        out_specs=[
            pl.BlockSpec((_TABLE_BLOCK,), lambda i: (i,)),
            pl.BlockSpec((_TABLE_BLOCK,), lambda i: (i,)),
        ],
        out_shape=[
            jax.ShapeDtypeStruct((v,), jnp.float32),
            jax.ShapeDtypeStruct((v,), jnp.float32),
        ],
    )(emb, ws_row, we_row, bs, be)


def _gather_logits(tab_s, tab_e, idx):
    n = idx.shape[0]
    n_per_tile = n // _NUM_SUBCORES
    assert n_per_tile % _CHUNK == 0 and _CHUNK % 16 == 0
    n_chunks = n_per_tile // _CHUNK
    mesh = plsc.VectorSubcoreMesh(
        core_axis_name="c",
        subcore_axis_name="s",
        num_cores=_NUM_SC,
        num_subcores=_NUM_SUBCORES,
    )

    @functools.partial(
        pl.kernel,
        out_type=(
            jax.ShapeDtypeStruct((n,), jnp.float32),
            jax.ShapeDtypeStruct((n,), jnp.float32),
        ),
        mesh=mesh,
        scratch_types=[
            pltpu.VMEM((tab_s.shape[0],), jnp.float32),
            pltpu.VMEM((_CHUNK,), jnp.int32),
            pltpu.VMEM((_CHUNK,), jnp.float32),
        ],
        compiler_params=pltpu.CompilerParams(needs_layout_passes=False),
    )
    def k(tabs_hbm, tabe_hbm, idx_hbm, outs_hbm, oute_hbm, tab_v, idx_v, out_v):
        c = lax.axis_index("c")
        s = lax.axis_index("s")
        base = s * n_per_tile

        @pl.when(c == 0)
        def _():
            pltpu.sync_copy(tabs_hbm, tab_v)

        @pl.when(c == 1)
        def _():
            pltpu.sync_copy(tabe_hbm, tab_v)

        for chunk in range(n_chunks):
            off = base + chunk * _CHUNK
            pltpu.sync_copy(idx_hbm.at[pl.ds(off, _CHUNK)], idx_v)

            def body(j, _):
                vals = plsc.load_gather(tab_v, [idx_v[pl.ds(j * 16, 16)]])
                out_v[pl.ds(j * 16, 16)] = vals
                return 0

            lax.fori_loop(0, _CHUNK // 16, body, 0)

            @pl.when(c == 0)
            def _():
                pltpu.sync_copy(out_v, outs_hbm.at[pl.ds(off, _CHUNK)])

            @pl.when(c == 1)
            def _():
                pltpu.sync_copy(out_v, oute_hbm.at[pl.ds(off, _CHUNK)])

    return k(tab_s, tab_e, idx)


def kernel(input_ids, emb, Ws, bs, We, be):
    b, l = input_ids.shape
    v = emb.shape[0]
    tab_s, tab_e = _build_tables(
        emb,
        Ws.reshape(1, -1),
        We.reshape(1, -1),
        bs.reshape(1, 1),
        be.reshape(1, 1),
    )
    idx = input_ids.reshape(-1).astype(jnp.int32)
    start_flat, end_flat = _gather_logits(tab_s, tab_e, idx)
    return (start_flat.reshape(b, l), end_flat.reshape(b, l))


# speedup vs baseline: 30.9085x; 1.0581x over previous
import functools

import jax
import jax.numpy as jnp
from jax import lax
from jax.experimental import pallas as pl
from jax.experimental.pallas import tpu as pltpu
from jax.experimental.pallas import tpu_sc as plsc

_NUM_SC = 2
_NUM_SUBCORES = 16

_TABLE_BLOCK = 16384
_CHUNK = 12800


def _table_body(emb_ref, ws_ref, we_ref, bs_ref, be_ref, outs_ref, oute_ref):
    emb_blk = emb_ref[...]
    dn = (((1,), (1,)), ((), ()))
    outs_ref[...] = (
        lax.dot_general(ws_ref[...], emb_blk, dn, preferred_element_type=jnp.float32)
        + bs_ref[0, 0]
    ).reshape(emb_blk.shape[0])
    oute_ref[...] = (
        lax.dot_general(we_ref[...], emb_blk, dn, preferred_element_type=jnp.float32)
        + be_ref[0, 0]
    ).reshape(emb_blk.shape[0])


def _build_tables(emb, ws_row, we_row, bs, be):
    v, h = emb.shape
    grid = pl.cdiv(v, _TABLE_BLOCK)
    return pl.pallas_call(
        _table_body,
        grid=(grid,),
        in_specs=[
            pl.BlockSpec((_TABLE_BLOCK, h), lambda i: (i, 0)),
            pl.BlockSpec((1, h), lambda i: (0, 0)),
            pl.BlockSpec((1, h), lambda i: (0, 0)),
            pl.BlockSpec((1, 1), lambda i: (0, 0)),
            pl.BlockSpec((1, 1), lambda i: (0, 0)),
        ],
        out_specs=[
            pl.BlockSpec((_TABLE_BLOCK,), lambda i: (i,)),
            pl.BlockSpec((_TABLE_BLOCK,), lambda i: (i,)),
        ],
        out_shape=[
            jax.ShapeDtypeStruct((v,), jnp.float32),
            jax.ShapeDtypeStruct((v,), jnp.float32),
        ],
    )(emb, ws_row, we_row, bs, be)


def _gather_logits(tab_s, tab_e, idx):
    n = idx.shape[0]
    n_per_tile = n // _NUM_SUBCORES
    assert n_per_tile % _CHUNK == 0 and _CHUNK % 16 == 0
    n_chunks = n_per_tile // _CHUNK
    mesh = plsc.VectorSubcoreMesh(
        core_axis_name="c",
        subcore_axis_name="s",
        num_cores=_NUM_SC,
        num_subcores=_NUM_SUBCORES,
    )

    @functools.partial(
        pl.kernel,
        out_type=(
            jax.ShapeDtypeStruct((n,), jnp.float32),
            jax.ShapeDtypeStruct((n,), jnp.float32),
        ),
        mesh=mesh,
        scratch_types=[
            pltpu.VMEM((tab_s.shape[0],), jnp.float32),
            pltpu.VMEM((_CHUNK,), jnp.int32),
            pltpu.VMEM((_CHUNK,), jnp.float32),
        ],
        compiler_params=pltpu.CompilerParams(needs_layout_passes=False),
    )
    def k(tabs_hbm, tabe_hbm, idx_hbm, outs_hbm, oute_hbm, tab_v, idx_v, out_v):
        c = lax.axis_index("c")
        s = lax.axis_index("s")
        base = s * n_per_tile

        @pl.when(c == 0)
        def _():
            pltpu.sync_copy(tabs_hbm, tab_v)

        @pl.when(c == 1)
        def _():
            pltpu.sync_copy(tabe_hbm, tab_v)

        for chunk in range(n_chunks):
            off = base + chunk * _CHUNK
            pltpu.sync_copy(idx_hbm.at[pl.ds(off, _CHUNK)], idx_v)

            def body(j, _):
                for k in range(4):
                    o = j * 64 + k * 16
                    vals = plsc.load_gather(tab_v, [idx_v[pl.ds(o, 16)]])
                    out_v[pl.ds(o, 16)] = vals
                return 0

            lax.fori_loop(0, _CHUNK // 64, body, 0)

            @pl.when(c == 0)
            def _():
                pltpu.sync_copy(out_v, outs_hbm.at[pl.ds(off, _CHUNK)])

            @pl.when(c == 1)
            def _():
                pltpu.sync_copy(out_v, oute_hbm.at[pl.ds(off, _CHUNK)])

    return k(tab_s, tab_e, idx)


def kernel(input_ids, emb, Ws, bs, We, be):
    b, l = input_ids.shape
    v = emb.shape[0]
    tab_s, tab_e = _build_tables(
        emb,
        Ws.reshape(1, -1),
        We.reshape(1, -1),
        bs.reshape(1, 1),
        be.reshape(1, 1),
    )
    idx = input_ids.reshape(-1).astype(jnp.int32)
    start_flat, end_flat = _gather_logits(tab_s, tab_e, idx)
    return (start_flat.reshape(b, l), end_flat.reshape(b, l))
